# baseline (speedup 1.0000x reference)
import functools

import jax
import jax.numpy as jnp
from jax import lax
from jax.experimental import pallas as pl
from jax.experimental.pallas import tpu as pltpu
from jax.experimental.pallas import tpu_sc as plsc

SEQ = 200
BATCH = 4096
EMB = 64
NC = 2
NS = 16
NW = NC * NS

TOTAL = SEQ * BATCH
PER_W = TOTAL // NW
K = 128
NCHUNK = PER_W // K


def _gather_body(x_hbm, table_hbm, out_hbm, idx_v, buf_v, sem):
    wid = lax.axis_index("s") * NC + lax.axis_index("c")
    base = wid * PER_W
    pltpu.sync_copy(x_hbm.at[wid], idx_v)

    def step(j, _):
        pltpu.async_copy(table_hbm.at[idx_v.at[j]], buf_v, sem).wait()
        pltpu.sync_copy(buf_v, out_hbm.at[pl.ds(base + j * K, K)])
        return 0

    lax.fori_loop(0, NCHUNK, step, 0)


@jax.jit
def kernel(x, table):
    x3 = x.astype(jnp.int32).reshape(NW, NCHUNK, K)
    out = pl.kernel(
        _gather_body,
        out_type=jax.ShapeDtypeStruct((TOTAL, EMB), jnp.float32),
        mesh=plsc.VectorSubcoreMesh(core_axis_name="c", subcore_axis_name="s"),
        scratch_types=[
            pltpu.VMEM((NCHUNK, K), jnp.int32),
            pltpu.VMEM((K, EMB), jnp.float32),
            pltpu.SemaphoreType.DMA,
        ],
        compiler_params=pltpu.CompilerParams(use_tc_tiling_on_sc=False),
    )(x3, table)
    return out.reshape(SEQ, BATCH, EMB)


# speedup vs baseline: 1.1175x; 1.1175x over previous
import functools

import jax
import jax.numpy as jnp
from jax import lax
from jax.experimental import pallas as pl
from jax.experimental.pallas import tpu as pltpu
from jax.experimental.pallas import tpu_sc as plsc

SEQ = 200
BATCH = 4096
EMB = 64
NC = 2
NS = 16
NW = NC * NS

TOTAL = SEQ * BATCH
PER_W = TOTAL // NW
K = 128
NCHUNK = PER_W // K


NBUF = 8
LOOKAHEAD = 4


def _gather_body(x_hbm, table_hbm, out_hbm, idx_v, bufs, gsem, osem):
    wid = lax.axis_index("s") * NC + lax.axis_index("c")
    base = wid * PER_W
    pltpu.sync_copy(x_hbm.at[wid], idx_v)

    def start_gather(j, b):
        pltpu.async_copy(table_hbm.at[idx_v.at[j]], bufs.at[b], gsem.at[b])

    def wait_gather(j, b):
        pltpu.make_async_copy(table_hbm.at[idx_v.at[j]], bufs.at[b],
                              gsem.at[b]).wait()

    def start_write(j, b):
        pltpu.async_copy(bufs.at[b], out_hbm.at[pl.ds(base + j * K, K)],
                         osem.at[b])

    def wait_write(j, b):
        pltpu.make_async_copy(bufs.at[b], out_hbm.at[pl.ds(base + j * K, K)],
                              osem.at[b]).wait()

    for b in range(LOOKAHEAD):
        start_gather(b, b)

    def outer(g, _):
        for bi in range(NBUF):
            j = g * NBUF + bi
            bn = (bi + LOOKAHEAD) % NBUF
            jp = j + LOOKAHEAD - NBUF

            @pl.when(jp >= 0)
            def _():
                wait_write(jp, bn)

            @pl.when(j + LOOKAHEAD < NCHUNK)
            def _():
                start_gather(j + LOOKAHEAD, bn)

            wait_gather(j, bi)
            start_write(j, bi)
        return 0

    lax.fori_loop(0, NCHUNK // NBUF, outer, 0)

    for t in range(LOOKAHEAD):
        j = NCHUNK - LOOKAHEAD + t
        wait_write(j, j % NBUF)


@jax.jit
def kernel(x, table):
    x3 = x.astype(jnp.int32).reshape(NW, NCHUNK, K)
    out = pl.kernel(
        _gather_body,
        out_type=jax.ShapeDtypeStruct((TOTAL, EMB), jnp.float32),
        mesh=plsc.VectorSubcoreMesh(core_axis_name="c", subcore_axis_name="s"),
        scratch_types=[
            pltpu.VMEM((NCHUNK, K), jnp.int32),
            pltpu.VMEM((NBUF, K, EMB), jnp.float32),
            pltpu.SemaphoreType.DMA((NBUF,)),
            pltpu.SemaphoreType.DMA((NBUF,)),
        ],
        compiler_params=pltpu.CompilerParams(use_tc_tiling_on_sc=False),
    )(x3, table)
    return out.reshape(SEQ, BATCH, EMB)
